# baseline (speedup 1.0000x reference)
import jax
import jax.numpy as jnp
from jax import lax
from jax.experimental import pallas as pl
from jax.experimental.pallas import tpu as pltpu
from jax.experimental.pallas import tpu_sc as plsc

NC = 2
NS = 16
LANE = 16
K = 128
CPT = 80
GRP = 8

_F32 = jnp.float32
_I32 = jnp.int32


def _mesh():
    return plsc.VectorSubcoreMesh(
        core_axis_name="c", subcore_axis_name="s", num_cores=NC, num_subcores=NS
    )


def _gather_rows(tab, idx):
    n_out = idx.shape[0]
    d = tab.shape[1]
    cpt = n_out // (NC * NS * K)
    idx_3d = idx.reshape(NC * NS, cpt, K)

    def body(tab_h, ih, oh, idxg, rows_a, rows_b, sem_a, sem_b):
        c = lax.axis_index("c")
        s = lax.axis_index("s")
        w = c * NS + s

        pltpu.sync_copy(ih.at[w], idxg)
        bufs = ((rows_a, sem_a), (rows_b, sem_b))

        def start(k):
            rows, sem = bufs[k % 2]
            pltpu.async_copy(tab_h.at[idxg.at[k]], rows, sem)

        def finish(k):
            rows, sem = bufs[k % 2]
            pltpu.make_async_copy(tab_h.at[idxg.at[k]], rows, sem).wait()
            pltpu.sync_copy(rows, oh.at[pl.ds((w * cpt + k) * K, K)])

        start(0)
        for k in range(1, cpt):
            start(k)
            finish(k - 1)
        finish(cpt - 1)

    return pl.kernel(
        body,
        out_type=jax.ShapeDtypeStruct((n_out, d), _F32),
        mesh=_mesh(),
        scratch_types=[
            pltpu.VMEM((cpt, K), _I32),
            pltpu.VMEM((K, d), _F32),
            pltpu.VMEM((K, d), _F32),
            pltpu.SemaphoreType.DMA,
            pltpu.SemaphoreType.DMA,
        ],
    )(tab, idx_3d)


def _edge_counts(col3, n_nodes):
    ngr = CPT // GRP
    rpt = n_nodes // NS
    cpr = rpt // K

    def body(col_h, ones_h, o_h,
             cg_a, cg_b, ones_v, buf, semg_a, semg_b, sem_a, sem_b, acc):
        c = lax.axis_index("c")
        s = lax.axis_index("s")

        pltpu.sync_copy(ones_h, ones_v)

        def zero_r(r, carry):
            def zero_l(l, carry2):
                buf[r, pl.ds(l * LANE, LANE)] = jnp.zeros((LANE,), _F32)
                return carry2
            return lax.fori_loop(0, K // LANE, zero_l, carry)
        lax.fori_loop(0, K, zero_r, 0)

        def zcp(k, carry):
            pltpu.sync_copy(buf, acc.at[pl.ds(s * rpt + k * K, K)])
            return carry
        lax.fori_loop(0, cpr, zcp, 0)
        plsc.subcore_barrier()

        base = s * CPT
        grp = ((cg_a, semg_a), (cg_b, semg_b))
        sems = (sem_a, sem_b)

        def load_group(g):
            cg, semg = grp[g % 2]
            pltpu.async_copy(col_h.at[c, pl.ds(base + g * GRP, GRP)], cg, semg)

        def wait_group(g):
            cg, semg = grp[g % 2]
            pltpu.make_async_copy(
                col_h.at[c, pl.ds(base + g * GRP, GRP)], cg, semg).wait()

        load_group(0)
        prev = None
        for g in range(ngr):
            cg, _ = grp[g % 2]
            wait_group(g)
            for t in range(GRP):
                k = g * GRP + t
                pltpu.async_copy(ones_v, acc.at[cg.at[t]], sems[k % 2],
                                 add=True)
                if prev is not None:
                    pcg, pt, psem = prev
                    pltpu.make_async_copy(
                        ones_v, acc.at[pcg.at[pt]], psem).wait()
                prev = (cg, t, sems[k % 2])
                if t == 0 and g + 1 < ngr:
                    load_group(g + 1)
        pcg, pt, psem = prev
        pltpu.make_async_copy(ones_v, acc.at[pcg.at[pt]], psem).wait()

        plsc.subcore_barrier()

        def out_cp(k, carry):
            r0 = s * rpt + k * K
            pltpu.sync_copy(acc.at[pl.ds(r0, K)], buf)
            pltpu.sync_copy(buf, o_h.at[c, pl.ds(r0, K)])
            return carry
        lax.fori_loop(0, cpr, out_cp, 0)

    return pl.kernel(
        body,
        out_type=jax.ShapeDtypeStruct((2, n_nodes, 128), _F32),
        mesh=_mesh(),
        scratch_types=[
            pltpu.VMEM((GRP, K), _I32),
            pltpu.VMEM((GRP, K), _I32),
            pltpu.VMEM((K, 128), _F32),
            pltpu.VMEM((K, 128), _F32),
            pltpu.SemaphoreType.DMA,
            pltpu.SemaphoreType.DMA,
            pltpu.SemaphoreType.DMA,
            pltpu.SemaphoreType.DMA,
            pltpu.VMEM_SHARED((n_nodes, 128), _F32),
        ],
    )(col3, jnp.ones((K, 128), _F32))


def _segsum(x2, ri3, col_2d, n_dst):
    ngr = CPT // GRP
    rpt = n_dst // NS
    hw = 128
    cpr = rpt // K

    def body(x2_h, ri_h, col_h, out_h,
             rg_a, cg_a, rg_b, cg_b, msg_a, msg_b,
             semg_a, semg_b, sem_a, sem_b, acc):
        c = lax.axis_index("c")
        s = lax.axis_index("s")

        def zero_r(r, carry):
            def zero_l(l, carry2):
                msg_a[r, pl.ds(l * LANE, LANE)] = jnp.zeros((LANE,), _F32)
                return carry2
            return lax.fori_loop(0, hw // LANE, zero_l, carry)
        lax.fori_loop(0, K, zero_r, 0)

        def zcp(k, carry):
            pltpu.sync_copy(msg_a, acc.at[pl.ds(s * rpt + k * K, K)])
            return carry
        lax.fori_loop(0, cpr, zcp, 0)
        plsc.subcore_barrier()

        base = s * CPT
        grp = ((rg_a, cg_a, semg_a), (rg_b, cg_b, semg_b))
        msgs = ((msg_a, sem_a), (msg_b, sem_b))

        def load_group(g):
            rg, cg, semg = grp[g % 2]
            pltpu.async_copy(ri_h.at[c, pl.ds(base + g * GRP, GRP)], rg, semg)
            pltpu.async_copy(col_h.at[pl.ds(base + g * GRP, GRP)], cg, semg)

        def wait_group(g):
            rg, cg, semg = grp[g % 2]
            pltpu.make_async_copy(
                ri_h.at[c, pl.ds(base + g * GRP, GRP)], rg, semg).wait()
            pltpu.make_async_copy(
                col_h.at[pl.ds(base + g * GRP, GRP)], cg, semg).wait()

        load_group(0)
        prev = None
        for g in range(ngr):
            rg, cg, _ = grp[g % 2]
            wait_group(g)
            for t in range(GRP):
                k = g * GRP + t
                msg, sem = msgs[k % 2]
                pltpu.async_copy(x2_h.at[rg.at[t]], msg, sem)
                if prev is not None:
                    pmsg, psem, prg, pt, pcg = prev
                    pltpu.make_async_copy(
                        x2_h.at[prg.at[pt]], pmsg, psem).wait()
                    pltpu.sync_copy(pmsg, acc.at[pcg.at[pt]], add=True)
                prev = (msg, sem, rg, t, cg)
                if t == 0 and g + 1 < ngr:
                    load_group(g + 1)
        pmsg, psem, prg, pt, pcg = prev
        pltpu.make_async_copy(x2_h.at[prg.at[pt]], pmsg, psem).wait()
        pltpu.sync_copy(pmsg, acc.at[pcg.at[pt]], add=True)

        plsc.subcore_barrier()

        def out_cp(k, carry):
            r0 = s * rpt + k * K
            pltpu.sync_copy(acc.at[pl.ds(r0, K)], msg_a)
            pltpu.sync_copy(msg_a, out_h.at[c, pl.ds(r0, K)])
            return carry
        lax.fori_loop(0, cpr, out_cp, 0)

    return pl.kernel(
        body,
        out_type=jax.ShapeDtypeStruct((2, n_dst, hw), _F32),
        mesh=_mesh(),
        scratch_types=[
            pltpu.VMEM((GRP, K), _I32),
            pltpu.VMEM((GRP, K), _I32),
            pltpu.VMEM((GRP, K), _I32),
            pltpu.VMEM((GRP, K), _I32),
            pltpu.VMEM((K, hw), _F32),
            pltpu.VMEM((K, hw), _F32),
            pltpu.SemaphoreType.DMA,
            pltpu.SemaphoreType.DMA,
            pltpu.SemaphoreType.DMA,
            pltpu.SemaphoreType.DMA,
            pltpu.VMEM_SHARED((n_dst, hw), _F32),
        ],
    )(x2, ri3, col_2d)


def _sage_update(s2, cnt, xdst, wlT, wrT, bl, relu):
    n = s2.shape[1]
    h = xdst.shape[1]
    br = 512
    grid = pl.cdiv(n, br)

    def body(slo, shi, c16, xd, wlo, whi, wr, b, o):
        cnt_col = c16[...][:, 0:1]
        rc = 1.0 / jnp.maximum(cnt_col, 1.0)
        acc = jnp.dot(slo[...] * rc, wlo[...],
                      preferred_element_type=_F32, precision=lax.Precision.HIGHEST)
        acc = acc + jnp.dot(shi[...] * rc, whi[...],
                            preferred_element_type=_F32, precision=lax.Precision.HIGHEST)
        acc = acc + jnp.dot(xd[...], wr[...],
                            preferred_element_type=_F32, precision=lax.Precision.HIGHEST)
        acc = acc + b[...]
        o[...] = jnp.maximum(acc, 0.0) if relu else acc

    return pl.pallas_call(
        body,
        grid=(grid,),
        in_specs=[
            pl.BlockSpec((br, 128), lambda i: (i, 0)),
            pl.BlockSpec((br, 128), lambda i: (i, 0)),
            pl.BlockSpec((br, 128), lambda i: (i, 0)),
            pl.BlockSpec((br, h), lambda i: (i, 0)),
            pl.BlockSpec((128, h), lambda i: (0, 0)),
            pl.BlockSpec((128, h), lambda i: (0, 0)),
            pl.BlockSpec((h, h), lambda i: (0, 0)),
            pl.BlockSpec((1, h), lambda i: (0, 0)),
        ],
        out_specs=pl.BlockSpec((br, h), lambda i: (i, 0)),
        out_shape=jax.ShapeDtypeStruct((n, h), _F32),
    )(s2[0], s2[1], cnt, xdst, wlT[:128], wlT[128:], wrT, bl.reshape(1, h))


def _decoder(zc, za, w1cT, w1aT, b1, w2, b2):
    lp = zc.shape[0]
    h = zc.shape[1]
    br = 512
    grid = lp // br

    def body(zc_r, za_r, wc, wa, b1r, w2r, b2r, o):
        hid = jnp.dot(zc_r[...], wc[...],
                      preferred_element_type=_F32, precision=lax.Precision.HIGHEST)
        hid = hid + jnp.dot(za_r[...], wa[...],
                            preferred_element_type=_F32, precision=lax.Precision.HIGHEST)
        hid = jnp.maximum(hid + b1r[...], 0.0)
        o[...] = jnp.sum(hid * w2r[...], axis=1) + b2r[0, 0]

    return pl.pallas_call(
        body,
        grid=(grid,),
        in_specs=[
            pl.BlockSpec((br, h), lambda i: (i, 0)),
            pl.BlockSpec((br, h), lambda i: (i, 0)),
            pl.BlockSpec((h, h), lambda i: (0, 0)),
            pl.BlockSpec((h, h), lambda i: (0, 0)),
            pl.BlockSpec((1, h), lambda i: (0, 0)),
            pl.BlockSpec((1, h), lambda i: (0, 0)),
            pl.BlockSpec((1, 1), lambda i: (0, 0)),
        ],
        out_specs=pl.BlockSpec((br,), lambda i: (i,)),
        out_shape=jax.ShapeDtypeStruct((lp,), _F32),
    )(zc, za, w1cT, w1aT, b1.reshape(1, h), w2, b2.reshape(1, 1))


def _pad_to(idx, n, fill=0):
    return jnp.concatenate(
        [idx.astype(_I32), jnp.full((n - idx.shape[0],), fill, _I32)])


def kernel(x_customer, x_article, edge_index_c2a, edge_index_a2c,
           edge_label_index, emb_customer, emb_article,
           wl1_ca, bl1_ca, wr1_ca, wl1_ac, bl1_ac, wr1_ac,
           wl2_ca, bl2_ca, wr2_ca, wl2_ac, bl2_ac, wr2_ac,
           w_dec1, b_dec1, w_dec2, b_dec2):
    n_c = x_customer.shape[0]
    n_a = x_article.shape[0]
    h = emb_customer.shape[1]
    n_lab = edge_label_index.shape[1]

    gran = NS * K
    np_node = pl.cdiv(max(n_c, n_a), gran) * gran
    lp = pl.cdiv(n_lab, gran) * gran
    epad = NC * NS * CPT * K
    trash = np_node - 1

    idx_c = _pad_to(x_customer[:, 0], np_node)
    idx_a = _pad_to(x_article[:, 0], np_node)
    emb2 = jnp.concatenate([emb_customer, emb_article])
    x_all = _gather_rows(emb2, jnp.concatenate([idx_c, idx_a + emb_customer.shape[0]]))
    xc_p, xa_p = x_all[:np_node], x_all[np_node:]

    row_a = edge_index_c2a[0].astype(_I32)
    col_a = edge_index_c2a[1].astype(_I32)
    row_c = edge_index_a2c[0].astype(_I32)
    col_c = edge_index_a2c[1].astype(_I32)

    ri_a = jnp.stack([_pad_to(row_a * 2, epad).reshape(-1, K),
                      _pad_to(row_a * 2 + 1, epad, 1).reshape(-1, K)])
    cp_a = _pad_to(col_a, epad, trash).reshape(-1, K)
    ri_c = jnp.stack([_pad_to(row_c * 2, epad).reshape(-1, K),
                      _pad_to(row_c * 2 + 1, epad, 1).reshape(-1, K)])
    cp_c = _pad_to(col_c, epad, trash).reshape(-1, K)

    cnt = _edge_counts(jnp.stack([cp_a, cp_c]), np_node)
    cnt_a, cnt_c = cnt[0], cnt[1]

    s_a1 = _segsum(xc_p.reshape(-1, 128), ri_a, cp_a, np_node)
    s_c1 = _segsum(xa_p.reshape(-1, 128), ri_c, cp_c, np_node)
    a1 = _sage_update(s_a1, cnt_a, xa_p, wl1_ca.T, wr1_ca.T, bl1_ca, relu=True)
    c1 = _sage_update(s_c1, cnt_c, xc_p, wl1_ac.T, wr1_ac.T, bl1_ac, relu=True)

    s_a2 = _segsum(c1.reshape(-1, 128), ri_a, cp_a, np_node)
    s_c2 = _segsum(a1.reshape(-1, 128), ri_c, cp_c, np_node)
    a2 = _sage_update(s_a2, cnt_a, a1, wl2_ca.T, wr2_ca.T, bl2_ca, relu=False)
    c2 = _sage_update(s_c2, cnt_c, c1, wl2_ac.T, wr2_ac.T, bl2_ac, relu=False)

    rowp = _pad_to(edge_label_index[0], lp)
    colp = _pad_to(edge_label_index[1], lp)
    ca2 = jnp.concatenate([c2, a2])
    z_all = _gather_rows(ca2, jnp.concatenate([rowp, colp + np_node]))
    zc, za = z_all[:lp], z_all[lp:]
    dec = _decoder(zc, za, w_dec1[:, :h].T, w_dec1[:, h:].T, b_dec1,
                   w_dec2, b_dec2)
    return dec[:n_lab]


# speedup vs baseline: 2.0307x; 2.0307x over previous
import jax
import jax.numpy as jnp
from jax import lax
from jax.experimental import pallas as pl
from jax.experimental.pallas import tpu as pltpu
from jax.experimental.pallas import tpu_sc as plsc

NC = 2
NS = 16
LANE = 16
K = 128

_F32 = jnp.float32
_I32 = jnp.int32


def _mesh():
    return plsc.VectorSubcoreMesh(
        core_axis_name="c", subcore_axis_name="s", num_cores=NC, num_subcores=NS
    )


def _dual_gather(tab0, idx0, tab1, idx1):
    n_out = idx0.shape[0]
    d = tab0.shape[1]
    cpt = n_out // (NS * K)

    def body(t0_h, i0_h, t1_h, i1_h, o0_h, o1_h,
             idx_a, rows_a, idx_b, rows_b,
             semi_a, semi_b, sem_a, sem_b):
        c = lax.axis_index("c")
        s = lax.axis_index("s")
        bufs = ((idx_a, semi_a, rows_a, sem_a), (idx_b, semi_b, rows_b, sem_b))

        def make(tab_h, ih, oh):
            def start_idx(i, p):
                idxv, semi, _, _ = bufs[p]

                @pl.when(i < cpt)
                def _():
                    pltpu.async_copy(
                        ih.at[pl.ds((s * cpt + i) * K, K)], idxv, semi)

            def start_gather(i, p):
                idxv, semi, rows, sem = bufs[p]

                @pl.when(i < cpt)
                def _():
                    pltpu.make_async_copy(
                        ih.at[pl.ds((s * cpt + i) * K, K)], idxv, semi).wait()
                    pltpu.async_copy(tab_h.at[idxv], rows, sem)

            def finish(i, p):
                idxv, _, rows, sem = bufs[p]

                @pl.when((i >= 0) & (i < cpt))
                def _():
                    pltpu.make_async_copy(tab_h.at[idxv], rows, sem).wait()
                    pltpu.sync_copy(rows, oh.at[pl.ds((s * cpt + i) * K, K)])

            def run():
                start_idx(0, 0)

                def chunkpair(i2, carry):
                    i0 = 2 * i2
                    start_idx(i0 + 1, 1)
                    start_gather(i0, 0)
                    finish(i0 - 1, 1)
                    start_idx(i0 + 2, 0)
                    start_gather(i0 + 1, 1)
                    finish(i0, 0)
                    return carry
                lax.fori_loop(0, (cpt + 1) // 2, chunkpair, 0)
                last = 2 * ((cpt + 1) // 2) - 1
                finish(last, last % 2)
            return run

        @pl.when(c == 0)
        def _():
            make(t0_h, i0_h, o0_h)()

        @pl.when(c == 1)
        def _():
            make(t1_h, i1_h, o1_h)()

    return pl.kernel(
        body,
        out_type=(
            jax.ShapeDtypeStruct((n_out, d), _F32),
            jax.ShapeDtypeStruct((n_out, d), _F32),
        ),
        mesh=_mesh(),
        scratch_types=[
            pltpu.VMEM((K,), _I32),
            pltpu.VMEM((K, d), _F32),
            pltpu.VMEM((K,), _I32),
            pltpu.VMEM((K, d), _F32),
            pltpu.SemaphoreType.DMA,
            pltpu.SemaphoreType.DMA,
            pltpu.SemaphoreType.DMA,
            pltpu.SemaphoreType.DMA,
        ],
    )(tab0, idx0, tab1, idx1)


def _edge_counts(col0, col1, n_nodes):
    e = col0.shape[0]
    nchunk = e // K
    iters = pl.cdiv(nchunk, NS)
    rpt = n_nodes // NS
    cpr = rpt // K

    def body(c0_h, c1_h, ones_h, o0_h, o1_h,
             colv_a, colv_b, ones_v, buf, semi_a, semi_b, sem_a, sem_b, acc):
        c = lax.axis_index("c")
        s = lax.axis_index("s")
        bufs = ((colv_a, semi_a, sem_a), (colv_b, semi_b, sem_b))

        pltpu.sync_copy(ones_h, ones_v)

        def zero_r(r, carry):
            def zero_l(l, carry2):
                buf[r, pl.ds(l * LANE, LANE)] = jnp.zeros((LANE,), _F32)
                return carry2
            return lax.fori_loop(0, K // LANE, zero_l, carry)
        lax.fori_loop(0, K, zero_r, 0)

        def zcp(k, carry):
            pltpu.sync_copy(buf, acc.at[pl.ds(s * rpt + k * K, K)])
            return carry
        lax.fori_loop(0, cpr, zcp, 0)
        plsc.subcore_barrier()

        def make(col_h):
            def jof(i):
                return s + NS * i

            def start_idx(i, p):
                colv, semi, _ = bufs[p]
                j = jof(i)

                @pl.when(j < nchunk)
                def _():
                    pltpu.async_copy(col_h.at[pl.ds(j * K, K)], colv, semi)

            def start_scat(i, p):
                colv, semi, sem = bufs[p]
                j = jof(i)

                @pl.when(j < nchunk)
                def _():
                    pltpu.make_async_copy(
                        col_h.at[pl.ds(j * K, K)], colv, semi).wait()
                    pltpu.async_copy(ones_v, acc.at[colv], sem, add=True)

            def finish(i, p):
                colv, _, sem = bufs[p]
                j = jof(i)

                @pl.when((i >= 0) & (j < nchunk))
                def _():
                    pltpu.make_async_copy(ones_v, acc.at[colv], sem).wait()

            def run():
                start_idx(0, 0)

                def chunkpair(i2, carry):
                    i0 = 2 * i2
                    start_idx(i0 + 1, 1)
                    start_scat(i0, 0)
                    finish(i0 - 1, 1)
                    start_idx(i0 + 2, 0)
                    start_scat(i0 + 1, 1)
                    finish(i0, 0)
                    return carry
                lax.fori_loop(0, (iters + 1) // 2, chunkpair, 0)
                last = 2 * ((iters + 1) // 2) - 1
                finish(last, last % 2)
            return run

        @pl.when(c == 0)
        def _():
            make(c0_h)()

        @pl.when(c == 1)
        def _():
            make(c1_h)()

        plsc.subcore_barrier()

        def out_cp(k, carry):
            r0 = s * rpt + k * K
            pltpu.sync_copy(acc.at[pl.ds(r0, K)], buf)

            @pl.when(c == 0)
            def _():
                pltpu.sync_copy(buf, o0_h.at[pl.ds(r0, K)])

            @pl.when(c == 1)
            def _():
                pltpu.sync_copy(buf, o1_h.at[pl.ds(r0, K)])
            return carry
        lax.fori_loop(0, cpr, out_cp, 0)

    return pl.kernel(
        body,
        out_type=(
            jax.ShapeDtypeStruct((n_nodes, 128), _F32),
            jax.ShapeDtypeStruct((n_nodes, 128), _F32),
        ),
        mesh=_mesh(),
        scratch_types=[
            pltpu.VMEM((K,), _I32),
            pltpu.VMEM((K,), _I32),
            pltpu.VMEM((K, 128), _F32),
            pltpu.VMEM((K, 128), _F32),
            pltpu.SemaphoreType.DMA,
            pltpu.SemaphoreType.DMA,
            pltpu.SemaphoreType.DMA,
            pltpu.SemaphoreType.DMA,
            pltpu.VMEM_SHARED((n_nodes, 128), _F32),
        ],
    )(col0, col1, jnp.ones((K, 128), _F32))


def _segsum(x2, row, col, n_dst):
    e = row.shape[0]
    nchunk = e // K
    iters = pl.cdiv(nchunk, NS)
    rpt = n_dst // NS
    hw = 128
    cpr = rpt // K

    def body(x2_h, row_h, col_h, out_h,
             rowv_a, row2v_a, colv_a, msg_a,
             rowv_b, row2v_b, colv_b, msg_b,
             semi_a, semi_b, sem_a, sem_b, acc):
        c = lax.axis_index("c")
        s = lax.axis_index("s")
        bufs = ((rowv_a, row2v_a, colv_a, msg_a, semi_a, sem_a),
                (rowv_b, row2v_b, colv_b, msg_b, semi_b, sem_b))

        def zero_r(r, carry):
            def zero_l(l, carry2):
                msg_a[r, pl.ds(l * LANE, LANE)] = jnp.zeros((LANE,), _F32)
                return carry2
            return lax.fori_loop(0, hw // LANE, zero_l, carry)
        lax.fori_loop(0, K, zero_r, 0)

        def zcp(k, carry):
            pltpu.sync_copy(msg_a, acc.at[pl.ds(s * rpt + k * K, K)])
            return carry
        lax.fori_loop(0, cpr, zcp, 0)
        plsc.subcore_barrier()

        def jof(i):
            return s + NS * i

        def start_idx(i, p):
            rowv, _, colv, _, semi, _ = bufs[p]
            j = jof(i)

            @pl.when(j < nchunk)
            def _():
                pltpu.async_copy(row_h.at[pl.ds(j * K, K)], rowv, semi)
                pltpu.async_copy(col_h.at[pl.ds(j * K, K)], colv, semi)

        def start_gather(i, p):
            rowv, row2v, colv, msg, semi, sem = bufs[p]
            j = jof(i)

            @pl.when(j < nchunk)
            def _():
                pltpu.make_async_copy(
                    row_h.at[pl.ds(j * K, K)], rowv, semi).wait()
                pltpu.make_async_copy(
                    col_h.at[pl.ds(j * K, K)], colv, semi).wait()

                def adj(k, carry2):
                    row2v[pl.ds(k * LANE, LANE)] = (
                        rowv[pl.ds(k * LANE, LANE)] * 2 + c
                    )
                    return carry2
                lax.fori_loop(0, K // LANE, adj, 0)
                pltpu.async_copy(x2_h.at[row2v], msg, sem)

        def finish(i, p):
            _, row2v, colv, msg, _, sem = bufs[p]
            j = jof(i)

            @pl.when((i >= 0) & (j < nchunk))
            def _():
                pltpu.make_async_copy(x2_h.at[row2v], msg, sem).wait()
                pltpu.sync_copy(msg, acc.at[colv], add=True)

        start_idx(0, 0)

        def chunkpair(i2, carry):
            i0 = 2 * i2
            start_idx(i0 + 1, 1)
            start_gather(i0, 0)
            finish(i0 - 1, 1)
            start_idx(i0 + 2, 0)
            start_gather(i0 + 1, 1)
            finish(i0, 0)
            return carry
        lax.fori_loop(0, (iters + 1) // 2, chunkpair, 0)
        last = 2 * ((iters + 1) // 2) - 1
        finish(last, last % 2)
        plsc.subcore_barrier()

        def out_cp(k, carry):
            r0 = s * rpt + k * K
            pltpu.sync_copy(acc.at[pl.ds(r0, K)], msg_a)

            @pl.when(c == 0)
            def _():
                pltpu.sync_copy(msg_a, out_h.at[0, pl.ds(r0, K)])

            @pl.when(c == 1)
            def _():
                pltpu.sync_copy(msg_a, out_h.at[1, pl.ds(r0, K)])
            return carry
        lax.fori_loop(0, cpr, out_cp, 0)

    return pl.kernel(
        body,
        out_type=jax.ShapeDtypeStruct((2, n_dst, hw), _F32),
        mesh=_mesh(),
        scratch_types=[
            pltpu.VMEM((K,), _I32),
            pltpu.VMEM((K,), _I32),
            pltpu.VMEM((K,), _I32),
            pltpu.VMEM((K, hw), _F32),
            pltpu.VMEM((K,), _I32),
            pltpu.VMEM((K,), _I32),
            pltpu.VMEM((K,), _I32),
            pltpu.VMEM((K, hw), _F32),
            pltpu.SemaphoreType.DMA,
            pltpu.SemaphoreType.DMA,
            pltpu.SemaphoreType.DMA,
            pltpu.SemaphoreType.DMA,
            pltpu.VMEM_SHARED((n_dst, hw), _F32),
        ],
    )(x2, row, col)


def _sage_update(s2, cnt, xdst, wlT, wrT, bl, relu):
    n = s2.shape[1]
    h = xdst.shape[1]
    br = 512
    grid = pl.cdiv(n, br)

    def body(slo, shi, c16, xd, wlo, whi, wr, b, o):
        cnt_col = c16[...][:, 0:1]
        rc = 1.0 / jnp.maximum(cnt_col, 1.0)
        acc = jnp.dot(slo[...] * rc, wlo[...],
                      preferred_element_type=_F32, precision=lax.Precision.HIGHEST)
        acc = acc + jnp.dot(shi[...] * rc, whi[...],
                            preferred_element_type=_F32, precision=lax.Precision.HIGHEST)
        acc = acc + jnp.dot(xd[...], wr[...],
                            preferred_element_type=_F32, precision=lax.Precision.HIGHEST)
        acc = acc + b[...]
        o[...] = jnp.maximum(acc, 0.0) if relu else acc

    return pl.pallas_call(
        body,
        grid=(grid,),
        in_specs=[
            pl.BlockSpec((br, 128), lambda i: (i, 0)),
            pl.BlockSpec((br, 128), lambda i: (i, 0)),
            pl.BlockSpec((br, 128), lambda i: (i, 0)),
            pl.BlockSpec((br, h), lambda i: (i, 0)),
            pl.BlockSpec((128, h), lambda i: (0, 0)),
            pl.BlockSpec((128, h), lambda i: (0, 0)),
            pl.BlockSpec((h, h), lambda i: (0, 0)),
            pl.BlockSpec((1, h), lambda i: (0, 0)),
        ],
        out_specs=pl.BlockSpec((br, h), lambda i: (i, 0)),
        out_shape=jax.ShapeDtypeStruct((n, h), _F32),
    )(s2[0], s2[1], cnt, xdst, wlT[:128], wlT[128:], wrT, bl.reshape(1, h))


def _decoder(zc, za, w1cT, w1aT, b1, w2, b2):
    lp = zc.shape[0]
    h = zc.shape[1]
    br = 512
    grid = lp // br

    def body(zc_r, za_r, wc, wa, b1r, w2r, b2r, o):
        hid = jnp.dot(zc_r[...], wc[...],
                      preferred_element_type=_F32, precision=lax.Precision.HIGHEST)
        hid = hid + jnp.dot(za_r[...], wa[...],
                            preferred_element_type=_F32, precision=lax.Precision.HIGHEST)
        hid = jnp.maximum(hid + b1r[...], 0.0)
        o[...] = jnp.sum(hid * w2r[...], axis=1) + b2r[0, 0]

    return pl.pallas_call(
        body,
        grid=(grid,),
        in_specs=[
            pl.BlockSpec((br, h), lambda i: (i, 0)),
            pl.BlockSpec((br, h), lambda i: (i, 0)),
            pl.BlockSpec((h, h), lambda i: (0, 0)),
            pl.BlockSpec((h, h), lambda i: (0, 0)),
            pl.BlockSpec((1, h), lambda i: (0, 0)),
            pl.BlockSpec((1, h), lambda i: (0, 0)),
            pl.BlockSpec((1, 1), lambda i: (0, 0)),
        ],
        out_specs=pl.BlockSpec((br,), lambda i: (i,)),
        out_shape=jax.ShapeDtypeStruct((lp,), _F32),
    )(zc, za, w1cT, w1aT, b1.reshape(1, h), w2, b2.reshape(1, 1))


def _pad_to(idx, n, fill=0):
    return jnp.concatenate(
        [idx.astype(_I32), jnp.full((n - idx.shape[0],), fill, _I32)])


def kernel(x_customer, x_article, edge_index_c2a, edge_index_a2c,
           edge_label_index, emb_customer, emb_article,
           wl1_ca, bl1_ca, wr1_ca, wl1_ac, bl1_ac, wr1_ac,
           wl2_ca, bl2_ca, wr2_ca, wl2_ac, bl2_ac, wr2_ac,
           w_dec1, b_dec1, w_dec2, b_dec2):
    n_c = x_customer.shape[0]
    n_a = x_article.shape[0]
    h = emb_customer.shape[1]
    n_lab = edge_label_index.shape[1]

    gran = NS * K
    np_node = pl.cdiv(max(n_c, n_a), gran) * gran
    lp = pl.cdiv(n_lab, gran) * gran

    idx_c = _pad_to(x_customer[:, 0], np_node)
    idx_a = _pad_to(x_article[:, 0], np_node)
    xc_p, xa_p = _dual_gather(emb_customer, idx_c, emb_article, idx_a)

    row_a = edge_index_c2a[0].astype(_I32)
    col_a = edge_index_c2a[1].astype(_I32)
    row_c = edge_index_a2c[0].astype(_I32)
    col_c = edge_index_a2c[1].astype(_I32)
    cnt_a, cnt_c = _edge_counts(col_a, col_c, np_node)

    s_a1 = _segsum(xc_p.reshape(-1, 128), row_a, col_a, np_node)
    s_c1 = _segsum(xa_p.reshape(-1, 128), row_c, col_c, np_node)
    a1 = _sage_update(s_a1, cnt_a, xa_p, wl1_ca.T, wr1_ca.T, bl1_ca, relu=True)
    c1 = _sage_update(s_c1, cnt_c, xc_p, wl1_ac.T, wr1_ac.T, bl1_ac, relu=True)

    s_a2 = _segsum(c1.reshape(-1, 128), row_a, col_a, np_node)
    s_c2 = _segsum(a1.reshape(-1, 128), row_c, col_c, np_node)
    a2 = _sage_update(s_a2, cnt_a, a1, wl2_ca.T, wr2_ca.T, bl2_ca, relu=False)
    c2 = _sage_update(s_c2, cnt_c, c1, wl2_ac.T, wr2_ac.T, bl2_ac, relu=False)

    rowp = _pad_to(edge_label_index[0], lp)
    colp = _pad_to(edge_label_index[1], lp)
    zc, za = _dual_gather(c2, rowp, a2, colp)
    dec = _decoder(zc, za, w_dec1[:, :h].T, w_dec1[:, h:].T, b_dec1,
                   w_dec2, b_dec2)
    return dec[:n_lab]
